# Initial kernel scaffold; baseline (speedup 1.0000x reference)
import functools

import jax
import jax.numpy as jnp
from jax import lax
from jax.experimental import pallas as pl
from jax.experimental.pallas import tpu as pltpu
from jax.experimental.pallas import tpu_sc as plsc

_B, _S, _D = 4, 2048, 768
_DFF = 1024
_E = 16
_T = _B * _S
_CAP = 1280
_PAD_ROW = _E * _CAP
_RB = 256
_NROWS = _PAD_ROW + _RB
_CT = 512
_G = _T // _CT

_NC, _NS = 2, 16
_NW = _NC * _NS
_TPW = _T // _NW
_CH = 64
_LANES = 16


def _router_body(x_ref, wg_ref, loc0_ref, loc1_ref, w0_ref, w1_ref, carry_ref):
    pid = pl.program_id(0)

    @pl.when(pid == 0)
    def _():
        carry_ref[...] = jnp.zeros_like(carry_ref)

    x = x_ref[...]
    wg = wg_ref[...]
    logits = jnp.dot(x, wg, preferred_element_type=jnp.float32)

    iota_e = lax.broadcasted_iota(jnp.int32, (_CT, _E), 1)
    m1 = jnp.max(logits, axis=1, keepdims=True)
    i1 = jnp.min(jnp.where(logits == m1, iota_e, _E), axis=1, keepdims=True)
    oh1 = iota_e == i1
    masked = jnp.where(oh1, -jnp.inf, logits)
    m2 = jnp.max(masked, axis=1, keepdims=True)
    i2 = jnp.min(jnp.where(masked == m2, iota_e, _E), axis=1, keepdims=True)
    oh2 = iota_e == i2

    g1 = 1.0 / (1.0 + jnp.exp(m2 - m1))
    g2 = 1.0 - g1

    ohsum = (oh1 | oh2).astype(jnp.float32)
    ri = lax.broadcasted_iota(jnp.int32, (_CT, _CT), 0)
    cj = lax.broadcasted_iota(jnp.int32, (_CT, _CT), 1)
    tril = (ri > cj).astype(jnp.float32)
    cex = jnp.dot(tril, ohsum, preferred_element_type=jnp.float32)
    base = carry_ref[0:1, 0:_E]
    p = cex + base
    pos1 = jnp.sum(jnp.where(oh1, p, 0.0), axis=1)
    pos2 = jnp.sum(jnp.where(oh2, p, 0.0), axis=1)
    carry_ref[0:1, 0:_E] = base + jnp.sum(ohsum, axis=0, keepdims=True)

    keep1 = pos1 < _CAP
    keep2 = pos2 < _CAP
    e1 = jnp.sum(jnp.where(oh1, iota_e, 0), axis=1)
    e2 = jnp.sum(jnp.where(oh2, iota_e, 0), axis=1)
    loc0_ref[...] = jnp.where(keep1, e1 * _CAP + pos1.astype(jnp.int32),
                              _PAD_ROW)
    loc1_ref[...] = jnp.where(keep2, e2 * _CAP + pos2.astype(jnp.int32),
                              _PAD_ROW)
    w0_ref[...] = jnp.where(keep1, g1[:, 0], 0.0)
    w1_ref[...] = jnp.where(keep2, g2[:, 0], 0.0)


_router = pl.pallas_call(
    _router_body,
    grid=(_G,),
    in_specs=[
        pl.BlockSpec((_CT, _D), lambda i: (i, 0)),
        pl.BlockSpec((_D, _E), lambda i: (0, 0)),
    ],
    out_specs=[
        pl.BlockSpec((_CT,), lambda i: (i,)),
        pl.BlockSpec((_CT,), lambda i: (i,)),
        pl.BlockSpec((_CT,), lambda i: (i,)),
        pl.BlockSpec((_CT,), lambda i: (i,)),
    ],
    out_shape=[
        jax.ShapeDtypeStruct((_T,), jnp.int32),
        jax.ShapeDtypeStruct((_T,), jnp.int32),
        jax.ShapeDtypeStruct((_T,), jnp.float32),
        jax.ShapeDtypeStruct((_T,), jnp.float32),
    ],
    scratch_shapes=[pltpu.VMEM((8, 128), jnp.float32)],
)


def _dispatch_body(xt_hbm, loc0_hbm, loc1_hbm, buf_hbm,
                   idx0_v, idx1_v, rows_v, sem):
    wid = lax.axis_index("s") * _NC + lax.axis_index("c")
    base = wid * _TPW
    for ci in range(_TPW // _CH):
        tok0 = base + ci * _CH
        pltpu.sync_copy(loc0_hbm.at[pl.ds(tok0, _CH)], idx0_v)
        pltpu.sync_copy(loc1_hbm.at[pl.ds(tok0, _CH)], idx1_v)
        pltpu.sync_copy(xt_hbm.at[pl.ds(tok0, _CH)], rows_v)
        pltpu.async_copy(rows_v, buf_hbm.at[idx0_v], sem).wait()
        pltpu.async_copy(rows_v, buf_hbm.at[idx1_v], sem).wait()


_dispatch = functools.partial(
    pl.kernel,
    out_type=jax.ShapeDtypeStruct((_NROWS, _D), jnp.float32),
    mesh=plsc.VectorSubcoreMesh(core_axis_name="c", subcore_axis_name="s"),
    scratch_types=[
        pltpu.VMEM((_CH,), jnp.int32),
        pltpu.VMEM((_CH,), jnp.int32),
        pltpu.VMEM((_CH, _D), jnp.float32),
        pltpu.SemaphoreType.DMA,
    ],
)(_dispatch_body)


def _ffn_body(buf_ref, w1_ref, b1_ref, w2_ref, b2_ref, y_ref):
    xb = buf_ref[...]
    h = jnp.dot(xb, w1_ref[0], preferred_element_type=jnp.float32)
    h = jax.nn.gelu(h + b1_ref[...])
    y = jnp.dot(h, w2_ref[0], preferred_element_type=jnp.float32)
    y_ref[...] = y + b2_ref[...]


def _expert_of(b):
    return jnp.minimum(b // (_CAP // _RB), _E - 1)


_ffn = pl.pallas_call(
    _ffn_body,
    grid=(_NROWS // _RB,),
    in_specs=[
        pl.BlockSpec((_RB, _D), lambda b: (b, 0)),
        pl.BlockSpec((1, _D, _DFF), lambda b: (_expert_of(b), 0, 0)),
        pl.BlockSpec((1, _DFF), lambda b: (_expert_of(b), 0)),
        pl.BlockSpec((1, _DFF, _D), lambda b: (_expert_of(b), 0, 0)),
        pl.BlockSpec((1, _D), lambda b: (_expert_of(b), 0)),
    ],
    out_specs=pl.BlockSpec((_RB, _D), lambda b: (b, 0)),
    out_shape=jax.ShapeDtypeStruct((_NROWS, _D), jnp.float32),
)


def _combine_body(y_hbm, loc0_hbm, loc1_hbm, w0_hbm, w1_hbm, out_hbm,
                  idx0_v, idx1_v, w0_v, w1_v, a_v, b_v, sem):
    wid = lax.axis_index("s") * _NC + lax.axis_index("c")
    base = wid * _TPW
    for ci in range(_TPW // _CH):
        tok0 = base + ci * _CH
        pltpu.sync_copy(loc0_hbm.at[pl.ds(tok0, _CH)], idx0_v)
        pltpu.sync_copy(loc1_hbm.at[pl.ds(tok0, _CH)], idx1_v)
        pltpu.sync_copy(w0_hbm.at[pl.ds(tok0, _CH)], w0_v)
        pltpu.sync_copy(w1_hbm.at[pl.ds(tok0, _CH)], w1_v)
        pltpu.async_copy(y_hbm.at[idx0_v], a_v, sem).wait()
        pltpu.async_copy(y_hbm.at[idx1_v], b_v, sem).wait()

        def tbody(t, _):
            tv = jnp.full((_LANES,), t, jnp.int32)
            wv0 = plsc.load_gather(w0_v, [tv])
            wv1 = plsc.load_gather(w1_v, [tv])
            for f in range(_D // _LANES):
                sl = pl.ds(f * _LANES, _LANES)
                a_v[t, sl] = a_v[t, sl] * wv0 + b_v[t, sl] * wv1
            return 0

        lax.fori_loop(0, _CH, tbody, 0)
        pltpu.sync_copy(a_v, out_hbm.at[pl.ds(tok0, _CH)])


_combine = functools.partial(
    pl.kernel,
    out_type=jax.ShapeDtypeStruct((_T, _D), jnp.float32),
    mesh=plsc.VectorSubcoreMesh(core_axis_name="c", subcore_axis_name="s"),
    scratch_types=[
        pltpu.VMEM((_CH,), jnp.int32),
        pltpu.VMEM((_CH,), jnp.int32),
        pltpu.VMEM((_CH,), jnp.float32),
        pltpu.VMEM((_CH,), jnp.float32),
        pltpu.VMEM((_CH, _D), jnp.float32),
        pltpu.VMEM((_CH, _D), jnp.float32),
        pltpu.SemaphoreType.DMA,
    ],
)(_combine_body)


def kernel(x, Wg, W1, b1, W2, b2):
    xt = x.reshape(_T, _D)
    loc0, loc1, w0, w1 = _router(xt, Wg)
    buf = _dispatch(xt, loc0, loc1)
    y = _ffn(buf, W1, b1, W2, b2)
    out = _combine(y, loc0, loc1, w0, w1)
    return out.reshape(_B, _S, _D)


# speedup vs baseline: 3.2456x; 3.2456x over previous
import functools

import jax
import jax.numpy as jnp
from jax import lax
from jax.experimental import pallas as pl
from jax.experimental.pallas import tpu as pltpu
from jax.experimental.pallas import tpu_sc as plsc

_B, _S, _D = 4, 2048, 768
_DFF = 1024
_E = 16
_T = _B * _S
_CAP = 1280
_PAD_ROW = _E * _CAP
_RB = 256
_NROWS = _PAD_ROW + _RB
_CT = 512
_G = _T // _CT

_NC, _NS = 2, 16
_NW = _NC * _NS
_TPW = _T // _NW
_CH = 64
_LANES = 16


def _router_body(x_ref, wg_ref, loc0_ref, loc1_ref, w0_ref, w1_ref, carry_ref):
    pid = pl.program_id(0)

    @pl.when(pid == 0)
    def _():
        carry_ref[...] = jnp.zeros_like(carry_ref)

    x = x_ref[...]
    wg = wg_ref[...]
    logits = jnp.dot(x, wg, preferred_element_type=jnp.float32)

    iota_e = lax.broadcasted_iota(jnp.int32, (_CT, _E), 1)
    m1 = jnp.max(logits, axis=1, keepdims=True)
    i1 = jnp.min(jnp.where(logits == m1, iota_e, _E), axis=1, keepdims=True)
    oh1 = iota_e == i1
    masked = jnp.where(oh1, -jnp.inf, logits)
    m2 = jnp.max(masked, axis=1, keepdims=True)
    i2 = jnp.min(jnp.where(masked == m2, iota_e, _E), axis=1, keepdims=True)
    oh2 = iota_e == i2

    g1 = 1.0 / (1.0 + jnp.exp(m2 - m1))
    g2 = 1.0 - g1

    ohsum = (oh1 | oh2).astype(jnp.float32)
    ri = lax.broadcasted_iota(jnp.int32, (_CT, _CT), 0)
    cj = lax.broadcasted_iota(jnp.int32, (_CT, _CT), 1)
    tril = (ri > cj).astype(jnp.float32)
    cex = jnp.dot(tril, ohsum, preferred_element_type=jnp.float32)
    base = carry_ref[0:1, 0:_E]
    p = cex + base
    pos1 = jnp.sum(jnp.where(oh1, p, 0.0), axis=1)
    pos2 = jnp.sum(jnp.where(oh2, p, 0.0), axis=1)
    carry_ref[0:1, 0:_E] = base + jnp.sum(ohsum, axis=0, keepdims=True)

    keep1 = pos1 < _CAP
    keep2 = pos2 < _CAP
    e1 = jnp.sum(jnp.where(oh1, iota_e, 0), axis=1)
    e2 = jnp.sum(jnp.where(oh2, iota_e, 0), axis=1)
    loc0_ref[...] = jnp.where(keep1, e1 * _CAP + pos1.astype(jnp.int32),
                              _PAD_ROW)
    loc1_ref[...] = jnp.where(keep2, e2 * _CAP + pos2.astype(jnp.int32),
                              _PAD_ROW)
    w0_ref[...] = jnp.where(keep1, g1[:, 0], 0.0)
    w1_ref[...] = jnp.where(keep2, g2[:, 0], 0.0)


_router = pl.pallas_call(
    _router_body,
    grid=(_G,),
    in_specs=[
        pl.BlockSpec((_CT, _D), lambda i: (i, 0)),
        pl.BlockSpec((_D, _E), lambda i: (0, 0)),
    ],
    out_specs=[
        pl.BlockSpec((_CT,), lambda i: (i,)),
        pl.BlockSpec((_CT,), lambda i: (i,)),
        pl.BlockSpec((_CT,), lambda i: (i,)),
        pl.BlockSpec((_CT,), lambda i: (i,)),
    ],
    out_shape=[
        jax.ShapeDtypeStruct((_T,), jnp.int32),
        jax.ShapeDtypeStruct((_T,), jnp.int32),
        jax.ShapeDtypeStruct((_T,), jnp.float32),
        jax.ShapeDtypeStruct((_T,), jnp.float32),
    ],
    scratch_shapes=[pltpu.VMEM((8, 128), jnp.float32)],
)


def _dispatch_body(xt_hbm, loc0_hbm, loc1_hbm, buf_hbm,
                   idx0_v, idx1_v, rows_v, sem):
    wid = lax.axis_index("s") * _NC + lax.axis_index("c")
    base = wid * _TPW
    for ci in range(_TPW // _CH):
        tok0 = base + ci * _CH
        pltpu.sync_copy(loc0_hbm.at[pl.ds(tok0, _CH)], idx0_v)
        pltpu.sync_copy(loc1_hbm.at[pl.ds(tok0, _CH)], idx1_v)
        pltpu.sync_copy(xt_hbm.at[pl.ds(tok0, _CH)], rows_v)
        pltpu.async_copy(rows_v, buf_hbm.at[idx0_v], sem).wait()
        pltpu.async_copy(rows_v, buf_hbm.at[idx1_v], sem).wait()


@functools.lru_cache(maxsize=None)
def _get_dispatch():
    return functools.partial(
        pl.kernel,
        out_type=jax.ShapeDtypeStruct((_NROWS, _D), jnp.float32),
        mesh=plsc.VectorSubcoreMesh(core_axis_name="c", subcore_axis_name="s",
                                    num_cores=_NC, num_subcores=_NS),
        scratch_types=[
            pltpu.VMEM((_CH,), jnp.int32),
            pltpu.VMEM((_CH,), jnp.int32),
            pltpu.VMEM((_CH, _D), jnp.float32),
            pltpu.SemaphoreType.DMA,
        ],
        compiler_params=pltpu.CompilerParams(needs_layout_passes=False),
    )(_dispatch_body)


def _ffn_body(buf_ref, w1_ref, b1_ref, w2_ref, b2_ref, y_ref):
    xb = buf_ref[...]
    h = jnp.dot(xb, w1_ref[0], preferred_element_type=jnp.float32)
    h = jax.nn.gelu(h + b1_ref[0])
    y = jnp.dot(h, w2_ref[0], preferred_element_type=jnp.float32)
    y_ref[...] = y + b2_ref[0]


def _expert_of(b):
    return jnp.minimum(b // (_CAP // _RB), _E - 1)


_ffn = pl.pallas_call(
    _ffn_body,
    grid=(_NROWS // _RB,),
    in_specs=[
        pl.BlockSpec((_RB, _D), lambda b: (b, 0)),
        pl.BlockSpec((1, _D, _DFF), lambda b: (_expert_of(b), 0, 0)),
        pl.BlockSpec((1, 1, _DFF), lambda b: (_expert_of(b), 0, 0)),
        pl.BlockSpec((1, _DFF, _D), lambda b: (_expert_of(b), 0, 0)),
        pl.BlockSpec((1, 1, _D), lambda b: (_expert_of(b), 0, 0)),
    ],
    out_specs=pl.BlockSpec((_RB, _D), lambda b: (b, 0)),
    out_shape=jax.ShapeDtypeStruct((_NROWS, _D), jnp.float32),
)


def _combine_body(y_hbm, loc0_hbm, loc1_hbm, w0_hbm, w1_hbm, out_hbm,
                  idx0_v, idx1_v, w0_v, w1_v, a_v, b_v, sem):
    wid = lax.axis_index("s") * _NC + lax.axis_index("c")
    base = wid * _TPW
    for ci in range(_TPW // _CH):
        tok0 = base + ci * _CH
        pltpu.sync_copy(loc0_hbm.at[pl.ds(tok0, _CH)], idx0_v)
        pltpu.sync_copy(loc1_hbm.at[pl.ds(tok0, _CH)], idx1_v)
        pltpu.sync_copy(w0_hbm.at[pl.ds(tok0, _CH)], w0_v)
        pltpu.sync_copy(w1_hbm.at[pl.ds(tok0, _CH)], w1_v)
        pltpu.async_copy(y_hbm.at[idx0_v], a_v, sem).wait()
        pltpu.async_copy(y_hbm.at[idx1_v], b_v, sem).wait()

        def tbody(t, _):
            tv = jnp.full((_LANES,), t, jnp.int32)
            wv0 = plsc.load_gather(w0_v, [tv])
            wv1 = plsc.load_gather(w1_v, [tv])
            for f in range(_D // _LANES):
                sl = pl.ds(f * _LANES, _LANES)
                a_v[t, sl] = a_v[t, sl] * wv0 + b_v[t, sl] * wv1
            return 0

        lax.fori_loop(0, _CH, tbody, 0)
        pltpu.sync_copy(a_v, out_hbm.at[pl.ds(tok0, _CH)])


@functools.lru_cache(maxsize=None)
def _get_combine():
    return functools.partial(
        pl.kernel,
        out_type=jax.ShapeDtypeStruct((_T, _D), jnp.float32),
        mesh=plsc.VectorSubcoreMesh(core_axis_name="c", subcore_axis_name="s",
                                    num_cores=_NC, num_subcores=_NS),
        scratch_types=[
            pltpu.VMEM((_CH,), jnp.int32),
            pltpu.VMEM((_CH,), jnp.int32),
            pltpu.VMEM((_CH,), jnp.float32),
            pltpu.VMEM((_CH,), jnp.float32),
            pltpu.VMEM((_CH, _D), jnp.float32),
            pltpu.VMEM((_CH, _D), jnp.float32),
            pltpu.SemaphoreType.DMA,
        ],
        compiler_params=pltpu.CompilerParams(needs_layout_passes=False),
    )(_combine_body)


def kernel(x, Wg, W1, b1, W2, b2):
    xt = x.reshape(_T, _D)
    loc0, loc1, w0, w1 = _router(xt, Wg)
    buf = _get_dispatch()(xt, loc0, loc1)
    y = _ffn(buf, W1, b1.reshape(_E, 1, _DFF), W2, b2.reshape(_E, 1, _D))
    out = _get_combine()(y, loc0, loc1, w0, w1)
    return out.reshape(_B, _S, _D)
